# Initial kernel scaffold; baseline (speedup 1.0000x reference)
import jax
import jax.numpy as jnp
from jax.experimental import pallas as pl


def kernel(feat, edge_index, W1, al1, ar1, b1, W2, al2, ar2, b2):
    raise NotImplementedError("write your pallas kernel here")


# speedup vs baseline: 41.8129x; 41.8129x over previous
import functools

import jax
import jax.numpy as jnp
from jax import lax
from jax.experimental import pallas as pl
from jax.experimental.pallas import tpu as pltpu
from jax.experimental.pallas import tpu_sc as plsc

_LANES = 16
_NCORES = 2
_NSUB = 16
_NTILES = _NCORES * _NSUB
_CHUNK = 128


def _dyn_gather16(vec, idx):
    return lax.gather(
        vec,
        idx[:, None],
        lax.GatherDimensionNumbers(
            offset_dims=(), collapsed_slice_dims=(0,), start_index_map=(0,)
        ),
        slice_sizes=(1,),
        mode=lax.GatherScatterMode.PROMISE_IN_BOUNDS,
    )


def _make_edge_kernel(epad, nv, gw, dh, expand_heads):
    per_tile = epad // _NTILES
    rows_per_tile = nv // _NSUB
    mesh = plsc.VectorSubcoreMesh(core_axis_name="c", subcore_axis_name="s")

    @functools.partial(
        pl.kernel,
        out_type=jax.ShapeDtypeStruct((_NCORES, nv, gw), jnp.float32),
        mesh=mesh,
        compiler_params=pltpu.CompilerParams(use_tc_tiling_on_sc=False),
        scratch_types=[
            pltpu.VMEM((_CHUNK, gw), jnp.float32),
            pltpu.VMEM((_CHUNK, 16), jnp.float32),
            pltpu.VMEM((_CHUNK, gw), jnp.float32),
            pltpu.VMEM((_CHUNK,), jnp.int32),
            pltpu.VMEM((_CHUNK,), jnp.int32),
            pltpu.VMEM_SHARED((nv, gw), jnp.float32),
            pltpu.SemaphoreType.DMA,
            pltpu.SemaphoreType.DMA,
        ],
    )
    def edge_kernel(hel_hbm, er_hbm, src_hbm, dst_hbm, out_hbm,
                    hbuf, erbuf, msg, sidx, didx, acc, sem1, sem2):
        cid = lax.axis_index("c")
        sid = lax.axis_index("s")
        wid = sid * _NCORES + cid
        zbase = sid * rows_per_tile

        @pl.loop(0, _CHUNK)
        def _(i):
            @pl.loop(0, gw, step=_LANES)
            def _(j):
                msg[i, pl.ds(j, _LANES)] = jnp.zeros((_LANES,), jnp.float32)

        zfull = (rows_per_tile // _CHUNK) * _CHUNK
        ztail = rows_per_tile % _CHUNK

        @pl.loop(0, zfull, step=_CHUNK)
        def _(r):
            pltpu.sync_copy(msg, acc.at[pl.ds(zbase + r, _CHUNK)])

        if ztail:
            pltpu.sync_copy(msg.at[pl.ds(0, ztail)],
                            acc.at[pl.ds(zbase + zfull, ztail)])

        plsc.subcore_barrier()

        if expand_heads:
            base_idx = jnp.right_shift(lax.iota(jnp.int32, _LANES), 3)
            exp_idx = [base_idx + (2 * j) for j in range(dh // _LANES)]

        ebase = wid * per_tile

        @pl.loop(0, per_tile, step=_CHUNK)
        def _(off):
            base = ebase + off
            pltpu.sync_copy(src_hbm.at[pl.ds(base, _CHUNK)], sidx)
            pltpu.sync_copy(dst_hbm.at[pl.ds(base, _CHUNK)], didx)
            g1 = pltpu.async_copy(hel_hbm.at[sidx], hbuf, sem1)
            g2 = pltpu.async_copy(er_hbm.at[didx], erbuf, sem2)
            g1.wait()
            g2.wait()

            @pl.loop(0, _CHUNK)
            def _(i):
                el = hbuf[i, pl.ds(dh, _LANES)]
                er = erbuf[i, pl.ds(0, _LANES)]
                t = el + er
                ex = jnp.exp(jnp.maximum(t, t * 0.2))
                msg[i, pl.ds(dh, _LANES)] = ex
                for j in range(dh // _LANES):
                    w = _dyn_gather16(ex, exp_idx[j]) if expand_heads else ex
                    msg[i, pl.ds(_LANES * j, _LANES)] = (
                        hbuf[i, pl.ds(_LANES * j, _LANES)] * w
                    )

            pltpu.sync_copy(msg, acc.at[didx], add=True)

        plsc.subcore_barrier()
        pltpu.sync_copy(
            acc.at[pl.ds(zbase, rows_per_tile)],
            out_hbm.at[cid, pl.ds(zbase, rows_per_tile)],
        )

    return edge_kernel


def _blockdiag(aflat, h, d):
    k = lax.broadcasted_iota(jnp.int32, (h * d, h), 0)
    j = lax.broadcasted_iota(jnp.int32, (h * d, h), 1)
    return jnp.where(k // d == j, aflat, 0.0)


def kernel(feat, edge_index, W1, al1, ar1, b1, W2, al2, ar2, b2):
    n, f_in = feat.shape
    e = edge_index.shape[1]
    h1, d1 = al1.shape
    hd = h1 * d1
    out_dim = W2.shape[1]
    gw1 = hd + 16
    gw2 = out_dim + 16

    nv = -(-(n + 8) // _CHUNK) * _CHUNK
    epad = -(-e // (_NTILES * _CHUNK)) * (_NTILES * _CHUNK)

    src = edge_index[0].astype(jnp.int32)
    dst = edge_index[1].astype(jnp.int32)
    npad_e = epad - e
    if npad_e:
        src = jnp.concatenate([src, jnp.full((npad_e,), n, jnp.int32)])
        dst = jnp.concatenate([dst, jnp.full((npad_e,), n, jnp.int32)])

    nblk = 8
    rb = nv // nblk
    featp = jnp.pad(feat, ((0, nv - n), (0, 0)))

    def _rows(w):
        return pl.BlockSpec((rb, w), lambda i: (i, 0))

    def _whole(*shape):
        nd = len(shape)
        return pl.BlockSpec(shape, lambda i, _z=(0,) * nd: _z)

    def proj1_body(feat_r, w1_r, al1_r, ar1_r, hel_r, erp_r):
        h = jnp.dot(feat_r[...], w1_r[...], preferred_element_type=jnp.float32)
        el = jnp.dot(h, _blockdiag(al1_r[...], h1, d1),
                     preferred_element_type=jnp.float32)
        er = jnp.dot(h, _blockdiag(ar1_r[...], h1, d1),
                     preferred_element_type=jnp.float32)
        zc = jnp.zeros((rb, 16 - h1), jnp.float32)
        hel_r[...] = jnp.concatenate([h, el, zc], axis=1)
        erp_r[...] = jnp.concatenate([er, zc], axis=1)

    hel1, er1p = pl.pallas_call(
        proj1_body,
        grid=(nblk,),
        in_specs=[_rows(f_in), _whole(f_in, hd), _whole(hd, 1), _whole(hd, 1)],
        out_specs=(_rows(gw1), _rows(16)),
        out_shape=(
            jax.ShapeDtypeStruct((nv, gw1), jnp.float32),
            jax.ShapeDtypeStruct((nv, 16), jnp.float32),
        ),
    )(featp, W1, al1.reshape(hd, 1), ar1.reshape(hd, 1))

    acc1 = _make_edge_kernel(epad, nv, gw1, hd, True)(hel1, er1p, src, dst)

    def mid_body(acc_r, hel_r, erp_r, b1_r, w2_r, al2_r, ar2_r,
                 hel2_r, er2_r):
        acc = acc_r[0] + acc_r[1]
        num = acc[:, :hd]
        den = acc[:, hd:hd + h1]
        h = hel_r[...][:, :hd]
        el = hel_r[...][:, hd:hd + h1]
        er = erp_r[...][:, :h1]
        t = el + er
        exs = jnp.exp(jnp.maximum(t, t * 0.2))
        num = num + jnp.repeat(exs, d1, axis=1) * h
        den = den + exs
        o1 = num / jnp.repeat(den, d1, axis=1) + b1_r[...]
        o1 = jnp.where(o1 > 0, o1, jnp.exp(jnp.minimum(o1, 0.0)) - 1.0)
        h2 = jnp.dot(o1, w2_r[...], preferred_element_type=jnp.float32)
        el2 = jnp.sum(h2 * al2_r[...], axis=1, keepdims=True)
        er2 = jnp.sum(h2 * ar2_r[...], axis=1, keepdims=True)
        hel2_r[...] = jnp.concatenate(
            [h2, jnp.broadcast_to(el2, (rb, 16))], axis=1)
        er2_r[...] = jnp.broadcast_to(er2, (rb, 16))

    hel2, er2p = pl.pallas_call(
        mid_body,
        grid=(nblk,),
        in_specs=[
            pl.BlockSpec((2, rb, gw1), lambda i: (0, i, 0)),
            _rows(gw1), _rows(16), _whole(1, hd), _whole(hd, out_dim),
            _whole(1, out_dim), _whole(1, out_dim),
        ],
        out_specs=(_rows(gw2), _rows(16)),
        out_shape=(
            jax.ShapeDtypeStruct((nv, gw2), jnp.float32),
            jax.ShapeDtypeStruct((nv, 16), jnp.float32),
        ),
    )(acc1, hel1, er1p, b1.reshape(1, hd), W2, al2, ar2)

    acc2 = _make_edge_kernel(epad, nv, gw2, out_dim, False)(hel2, er2p, src, dst)

    def final_body(acc_r, hel2_r, er2_r, b2_r, out_r):
        acc = acc_r[0] + acc_r[1]
        num = acc[:, :out_dim]
        den = acc[:, out_dim:out_dim + 1]
        h2 = hel2_r[...][:, :out_dim]
        el2 = hel2_r[...][:, out_dim:out_dim + 1]
        er2 = er2_r[...][:, :1]
        t = el2 + er2
        exs = jnp.exp(jnp.maximum(t, t * 0.2))
        out_r[...] = (num + exs * h2) / (den + exs) + b2_r[...]

    return pl.pallas_call(
        final_body,
        grid=(nblk,),
        in_specs=[
            pl.BlockSpec((2, rb, gw2), lambda i: (0, i, 0)),
            _rows(gw2), _rows(16), _whole(1, out_dim),
        ],
        out_specs=_rows(out_dim),
        out_shape=jax.ShapeDtypeStruct((n, out_dim), jnp.float32),
    )(acc2, hel2, er2p, b2.reshape(1, out_dim))
